# baseline (speedup 1.0000x reference)
import functools

import jax
import jax.numpy as jnp
from jax import lax
from jax.experimental import pallas as pl
from jax.experimental.pallas import tpu as pltpu
from jax.experimental.pallas import tpu_sc as plsc

E = 320000
D_IN = 128
D_OUT = 256
G = 512
EPS = 1e-5

NC = 2
NS = 16
NW = NC * NS
NCHUNK = 16
Q = 5
ES = E // Q
EPC = ES // NCHUNK
B = 2000
NGRP = B // 16
NH = D_OUT // 128
HL = 128
NV = HL // 16
EG = ES // 16


def _mm_body(x_ref, w_ref, b_ref, h_ref, g_ref):
    h = jnp.dot(x_ref[...], w_ref[...], preferred_element_type=jnp.float32)
    h = jnp.maximum(h + b_ref[...], 0.0)
    h_ref[0] = h[:, :HL]
    h_ref[1] = h[:, HL:]
    g = jnp.sum(h.reshape(h.shape[0] // 16, 16, D_OUT), axis=1)
    g_ref[0] = g[:, :HL]
    g_ref[1] = g[:, HL:]


def _matmul_relu(edge_attr, W, b):
    BM = 3200
    return pl.pallas_call(
        _mm_body,
        grid=(ES // BM,),
        in_specs=[
            pl.BlockSpec((BM, D_IN), lambda i: (i, 0)),
            pl.BlockSpec((D_IN, D_OUT), lambda i: (0, 0)),
            pl.BlockSpec((1, D_OUT), lambda i: (0, 0)),
        ],
        out_specs=[
            pl.BlockSpec((NH, BM, HL), lambda i: (0, i, 0)),
            pl.BlockSpec((NH, BM // 16, HL), lambda i: (0, i, 0)),
        ],
        out_shape=[
            jax.ShapeDtypeStruct((NH, ES, HL), jnp.float32),
            jax.ShapeDtypeStruct((NH, EG, HL), jnp.float32),
        ],
    )(edge_attr, W, b.reshape(1, D_OUT))


def _seg_body(h_hbm, g_hbm, batch_hbm, psum_hbm, pcnt_hbm, acc, cntacc,
              gstage0, gstage1, bstage0, bstage1, hslot,
              gsem0, gsem1, bsem0, bsem1, hsem):
    wid = lax.axis_index("s") * NC + lax.axis_index("c")
    chunk = wid // NH
    fh = wid % NH
    base = chunk * EPC
    hbase = fh * (ES * HL) + base * HL
    gbase = fh * (EG * HL) + (base // 16) * HL
    bufs = ((gstage0, bstage0, gsem0, bsem0),
            (gstage1, bstage1, gsem1, bsem1))

    def issue(t, buf):
        gstage, bstage, gsem, bsem = bufs[buf]
        pltpu.make_async_copy(
            batch_hbm.at[pl.ds(base + t * B, B)], bstage, bsem).start()
        pltpu.make_async_copy(
            g_hbm.at[pl.ds(gbase + t * NGRP * HL, NGRP * HL)],
            gstage, gsem).start()

    def wait(buf):
        gstage, bstage, gsem, bsem = bufs[buf]
        pltpu.make_async_copy(
            batch_hbm.at[pl.ds(base, B)], bstage, bsem).wait()
        pltpu.make_async_copy(
            g_hbm.at[pl.ds(gbase, NGRP * HL)], gstage, gsem).wait()

    @pl.loop(0, (G + 1) * HL, step=16)
    def _(g):
        acc[pl.ds(g, 16)] = jnp.zeros((16,), jnp.float32)

    @pl.loop(0, (G + 1) * 16, step=16)
    def _(g):
        cntacc[pl.ds(g, 16)] = jnp.zeros((16,), jnp.float32)

    def make_group_body(t, gstage, bstage):
        def group_body(g, carry):
            segvec = bstage[pl.ds(16 * g, 16)]
            sfirst = segvec[0]
            slast = segvec[15]
            prev, cntvec = carry[0], carry[1]
            a = list(carry[2:])
            uniform = sfirst == slast
            run_ends = jnp.logical_not(jnp.logical_and(uniform,
                                                       sfirst == prev))

            @pl.when(run_ends)
            def _():
                for j in range(NV):
                    p = pl.ds(prev * HL + 16 * j, 16)
                    acc[p] = acc[p] + a[j]
                q = pl.ds(prev * 16, 16)
                cntacc[q] = cntacc[q] + cntvec

            @pl.when(jnp.logical_not(uniform))
            def _():
                pltpu.sync_copy(
                    h_hbm.at[pl.ds(hbase + (t * B + 16 * g) * HL, 16 * HL)],
                    hslot)
                for k in range(16):
                    seg = segvec[k]
                    for j in range(NV):
                        p = pl.ds(seg * HL + 16 * j, 16)
                        acc[p] = acc[p] + hslot[pl.ds(k * HL + 16 * j, 16)]
                    q = pl.ds(seg * 16, 16)
                    cntacc[q] = cntacc[q] + 1.0

            kuni = jnp.where(uniform, 1.0, 0.0)
            kcont = jnp.where(run_ends, 0.0, 1.0)
            for j in range(NV):
                a[j] = (a[j] * kcont
                        + gstage[pl.ds(g * HL + 16 * j, 16)]) * kuni
            cntvec = (cntvec * kcont + 16.0) * kuni
            return (slast, cntvec) + tuple(a)
        return group_body

    NT = EPC // B

    def pair_body(tt, carry):
        t0 = 2 * tt

        wait(0)
        carry = lax.fori_loop(0, NGRP,
                              make_group_body(t0, gstage0, bstage0), carry)

        @pl.when(t0 + 2 < NT)
        def _():
            issue(t0 + 2, 0)

        wait(1)
        carry = lax.fori_loop(0, NGRP,
                              make_group_body(t0 + 1, gstage1, bstage1),
                              carry)

        @pl.when(t0 + 3 < NT)
        def _():
            issue(t0 + 3, 1)

        return carry

    issue(0, 0)
    issue(1, 1)
    init = (jnp.int32(G), jnp.zeros((16,), jnp.float32)) + tuple(
        jnp.zeros((16,), jnp.float32) for _ in range(NV))
    final = lax.fori_loop(0, NT // 2, pair_body, init)

    prev, cntvec = final[0], final[1]
    for j in range(NV):
        p = pl.ds(prev * HL + 16 * j, 16)
        acc[p] = acc[p] + final[2 + j]
    q = pl.ds(prev * 16, 16)
    cntacc[q] = cntacc[q] + cntvec

    pltpu.sync_copy(acc.at[pl.ds(0, G * HL)],
                    psum_hbm.at[pl.ds(wid * G * HL, G * HL)])

    @pl.when(fh == 0)
    def _():
        pltpu.sync_copy(cntacc.at[pl.ds(0, G * 16)], pcnt_hbm.at[chunk])


def _segsum(h_flat, g_flat, batch_slice):
    mesh = plsc.VectorSubcoreMesh(core_axis_name="c", subcore_axis_name="s")
    f = pl.kernel(
        _seg_body,
        out_type=(
            jax.ShapeDtypeStruct((NW * G * HL,), jnp.float32),
            jax.ShapeDtypeStruct((NCHUNK, G * 16), jnp.float32),
        ),
        mesh=mesh,
        scratch_types=[
            pltpu.VMEM(((G + 1) * HL,), jnp.float32),
            pltpu.VMEM(((G + 1) * 16,), jnp.float32),
            pltpu.VMEM((NGRP * HL,), jnp.float32),
            pltpu.VMEM((NGRP * HL,), jnp.float32),
            pltpu.VMEM((B,), jnp.int32),
            pltpu.VMEM((B,), jnp.int32),
            pltpu.VMEM((16 * HL,), jnp.float32),
            pltpu.SemaphoreType.DMA,
            pltpu.SemaphoreType.DMA,
            pltpu.SemaphoreType.DMA,
            pltpu.SemaphoreType.DMA,
            pltpu.SemaphoreType.DMA,
        ],
    )
    psum, pcnt = f(h_flat, g_flat, batch_slice)
    return psum.reshape(NW, G, HL), pcnt.reshape(NCHUNK, G, 16)


def _ln_body(ps_ref, pc_ref, lnw_ref, lnb_ref, o_ref, sacc, cacc):
    qi = pl.program_id(0)

    s0 = ps_ref[0, 0]
    s1 = ps_ref[0, 1]
    for c in range(1, NCHUNK):
        s0 = s0 + ps_ref[0, NH * c]
        s1 = s1 + ps_ref[0, NH * c + 1]
    s = jnp.concatenate([s0, s1], axis=1)
    cnt = jnp.sum(pc_ref[0], axis=(0, 2)) * (1.0 / 16.0)

    @pl.when(qi == 0)
    def _():
        sacc[...] = s
        cacc[...] = cnt.reshape(1, G)

    @pl.when(qi > 0)
    def _():
        sacc[...] = sacc[...] + s
        cacc[...] = cacc[...] + cnt.reshape(1, G)

    @pl.when(qi == Q - 1)
    def _():
        tot = sacc[...]
        cnt_all = cacc[...].reshape(G)
        mean_g = tot / jnp.clip(cnt_all, 1.0)[:, None]
        mu = jnp.mean(mean_g, axis=-1, keepdims=True)
        var = jnp.mean((mean_g - mu) ** 2, axis=-1, keepdims=True)
        o_ref[...] = ((mean_g - mu) * lax.rsqrt(var + EPS) * lnw_ref[...]
                      + lnb_ref[...])


def _layernorm(psums, pcnts, ln_w, ln_b):
    return pl.pallas_call(
        _ln_body,
        grid=(Q,),
        in_specs=[
            pl.BlockSpec((1, NW, G, HL), lambda qi: (qi, 0, 0, 0)),
            pl.BlockSpec((1, NCHUNK, G, 16), lambda qi: (qi, 0, 0, 0)),
            pl.BlockSpec((1, D_OUT), lambda qi: (0, 0)),
            pl.BlockSpec((1, D_OUT), lambda qi: (0, 0)),
        ],
        out_specs=pl.BlockSpec((G, D_OUT), lambda qi: (0, 0)),
        out_shape=jax.ShapeDtypeStruct((G, D_OUT), jnp.float32),
        scratch_shapes=[
            pltpu.VMEM((G, D_OUT), jnp.float32),
            pltpu.VMEM((1, G), jnp.float32),
        ],
    )(jnp.stack(psums), jnp.stack(pcnts),
      ln_w.reshape(1, D_OUT), ln_b.reshape(1, D_OUT))


def kernel(edge_attr, batch, W, b, ln_w, ln_b):
    batch = batch.astype(jnp.int32)
    psums, pcnts = [], []
    for qi in range(Q):
        h, gsum = _matmul_relu(edge_attr[qi * ES:(qi + 1) * ES], W, b)
        psum, pcnt = _segsum(h.reshape(NH * ES * HL),
                             gsum.reshape(NH * EG * HL),
                             batch[qi * ES:(qi + 1) * ES])
        psums.append(psum)
        pcnts.append(pcnt)
    return _layernorm(psums, pcnts, ln_w, ln_b)


# speedup vs baseline: 1.4953x; 1.4953x over previous
import functools

import jax
import jax.numpy as jnp
from jax import lax
from jax.experimental import pallas as pl
from jax.experimental.pallas import tpu as pltpu
from jax.experimental.pallas import tpu_sc as plsc

E = 320000
D_IN = 128
D_OUT = 256
G = 512
EPS = 1e-5

NC = 2
NS = 16
NW = NC * NS
NCHUNK = 16
EPC = E // NCHUNK
B = 2000
NGRP = B // 16
NH = D_OUT // 128
HL = 128
NV = HL // 16
EG = E // 16


def _mm_body(x_ref, w_ref, b_ref, h_ref, g_ref):
    h = jnp.dot(x_ref[...], w_ref[...], preferred_element_type=jnp.float32)
    h = jnp.maximum(h + b_ref[...], 0.0)
    h_ref[0] = h[:, :HL]
    h_ref[1] = h[:, HL:]
    g = jnp.sum(h.reshape(h.shape[0] // 16, 16, D_OUT), axis=1)
    g_ref[0] = g[:, :HL]
    g_ref[1] = g[:, HL:]


def _matmul_relu(edge_attr, W, b):
    BM = 3200
    return pl.pallas_call(
        _mm_body,
        grid=(E // BM,),
        in_specs=[
            pl.BlockSpec((BM, D_IN), lambda i: (i, 0)),
            pl.BlockSpec((D_IN, D_OUT), lambda i: (0, 0)),
            pl.BlockSpec((1, D_OUT), lambda i: (0, 0)),
        ],
        out_specs=[
            pl.BlockSpec((NH, BM, HL), lambda i: (0, i, 0)),
            pl.BlockSpec((NH, BM // 16, HL), lambda i: (0, i, 0)),
        ],
        out_shape=[
            jax.ShapeDtypeStruct((NH, E, HL), jnp.float32),
            jax.ShapeDtypeStruct((NH, EG, HL), jnp.float32),
        ],
    )(edge_attr, W, b.reshape(1, D_OUT))


def _seg_body(h_hbm, g_hbm, batch_hbm, psum_hbm, pcnt_hbm, acc, cntacc,
              gstage0, gstage1, bstage0, bstage1, hslot,
              gsem0, gsem1, bsem0, bsem1, hsem):
    wid = lax.axis_index("s") * NC + lax.axis_index("c")
    chunk = wid // NH
    fh = wid % NH
    base = chunk * EPC
    hbase = fh * (E * HL) + base * HL
    gbase = fh * (EG * HL) + (base // 16) * HL
    bufs = ((gstage0, bstage0, gsem0, bsem0),
            (gstage1, bstage1, gsem1, bsem1))

    def issue(t, buf):
        gstage, bstage, gsem, bsem = bufs[buf]
        pltpu.make_async_copy(
            batch_hbm.at[pl.ds(base + t * B, B)], bstage, bsem).start()
        pltpu.make_async_copy(
            g_hbm.at[pl.ds(gbase + t * NGRP * HL, NGRP * HL)],
            gstage, gsem).start()

    def wait(buf):
        gstage, bstage, gsem, bsem = bufs[buf]
        pltpu.make_async_copy(
            batch_hbm.at[pl.ds(base, B)], bstage, bsem).wait()
        pltpu.make_async_copy(
            g_hbm.at[pl.ds(gbase, NGRP * HL)], gstage, gsem).wait()

    @pl.loop(0, (G + 1) * HL, step=16)
    def _(g):
        acc[pl.ds(g, 16)] = jnp.zeros((16,), jnp.float32)

    @pl.loop(0, (G + 1) * 16, step=16)
    def _(g):
        cntacc[pl.ds(g, 16)] = jnp.zeros((16,), jnp.float32)

    def make_group_body(t, gstage, bstage):
        def group_body(g, carry):
            segvec = bstage[pl.ds(16 * g, 16)]
            sfirst = segvec[0]
            slast = segvec[15]
            prev, cntvec = carry[0], carry[1]
            a = list(carry[2:])
            uniform = sfirst == slast
            run_ends = jnp.logical_not(jnp.logical_and(uniform,
                                                       sfirst == prev))

            @pl.when(run_ends)
            def _():
                for j in range(NV):
                    p = pl.ds(prev * HL + 16 * j, 16)
                    acc[p] = acc[p] + a[j]
                q = pl.ds(prev * 16, 16)
                cntacc[q] = cntacc[q] + cntvec

            @pl.when(jnp.logical_not(uniform))
            def _():
                pltpu.sync_copy(
                    h_hbm.at[pl.ds(hbase + (t * B + 16 * g) * HL, 16 * HL)],
                    hslot)
                for k in range(16):
                    seg = segvec[k]
                    for j in range(NV):
                        p = pl.ds(seg * HL + 16 * j, 16)
                        acc[p] = acc[p] + hslot[pl.ds(k * HL + 16 * j, 16)]
                    q = pl.ds(seg * 16, 16)
                    cntacc[q] = cntacc[q] + 1.0

            kuni = jnp.where(uniform, 1.0, 0.0)
            kcont = jnp.where(run_ends, 0.0, 1.0)
            for j in range(NV):
                a[j] = (a[j] * kcont
                        + gstage[pl.ds(g * HL + 16 * j, 16)]) * kuni
            cntvec = (cntvec * kcont + 16.0) * kuni
            return (slast, cntvec) + tuple(a)
        return group_body

    NT = EPC // B

    def pair_body(tt, carry):
        t0 = 2 * tt

        wait(0)
        carry = lax.fori_loop(0, NGRP,
                              make_group_body(t0, gstage0, bstage0), carry)

        @pl.when(t0 + 2 < NT)
        def _():
            issue(t0 + 2, 0)

        wait(1)
        carry = lax.fori_loop(0, NGRP,
                              make_group_body(t0 + 1, gstage1, bstage1),
                              carry)

        @pl.when(t0 + 3 < NT)
        def _():
            issue(t0 + 3, 1)

        return carry

    issue(0, 0)
    issue(1, 1)
    init = (jnp.int32(G), jnp.zeros((16,), jnp.float32)) + tuple(
        jnp.zeros((16,), jnp.float32) for _ in range(NV))
    final = lax.fori_loop(0, NT // 2, pair_body, init)

    prev, cntvec = final[0], final[1]
    for j in range(NV):
        p = pl.ds(prev * HL + 16 * j, 16)
        acc[p] = acc[p] + final[2 + j]
    q = pl.ds(prev * 16, 16)
    cntacc[q] = cntacc[q] + cntvec

    pltpu.sync_copy(acc.at[pl.ds(0, G * HL)],
                    psum_hbm.at[pl.ds(wid * G * HL, G * HL)])

    @pl.when(fh == 0)
    def _():
        pltpu.sync_copy(cntacc.at[pl.ds(0, G * 16)], pcnt_hbm.at[chunk])


def _segsum(h_flat, g_flat, batch):
    mesh = plsc.VectorSubcoreMesh(core_axis_name="c", subcore_axis_name="s")
    f = pl.kernel(
        _seg_body,
        out_type=(
            jax.ShapeDtypeStruct((NW * G * HL,), jnp.float32),
            jax.ShapeDtypeStruct((NCHUNK, G * 16), jnp.float32),
        ),
        mesh=mesh,
        scratch_types=[
            pltpu.VMEM(((G + 1) * HL,), jnp.float32),
            pltpu.VMEM(((G + 1) * 16,), jnp.float32),
            pltpu.VMEM((NGRP * HL,), jnp.float32),
            pltpu.VMEM((NGRP * HL,), jnp.float32),
            pltpu.VMEM((B,), jnp.int32),
            pltpu.VMEM((B,), jnp.int32),
            pltpu.VMEM((16 * HL,), jnp.float32),
            pltpu.SemaphoreType.DMA,
            pltpu.SemaphoreType.DMA,
            pltpu.SemaphoreType.DMA,
            pltpu.SemaphoreType.DMA,
            pltpu.SemaphoreType.DMA,
        ],
    )
    psum, pcnt = f(h_flat, g_flat, batch)
    return psum.reshape(NW, G, HL), pcnt.reshape(NCHUNK, G, 16)


def _ln_body(ps_ref, pc_ref, lnw_ref, lnb_ref, o_ref):
    s0 = ps_ref[0]
    s1 = ps_ref[1]
    for c in range(1, NCHUNK):
        s0 = s0 + ps_ref[NH * c]
        s1 = s1 + ps_ref[NH * c + 1]
    cnt = jnp.sum(pc_ref[...], axis=(0, 2)) * (1.0 / 16.0)
    mean_g = jnp.concatenate([s0, s1], axis=1) / jnp.clip(cnt, 1.0)[:, None]
    mu = jnp.mean(mean_g, axis=-1, keepdims=True)
    var = jnp.mean((mean_g - mu) ** 2, axis=-1, keepdims=True)
    o_ref[...] = ((mean_g - mu) * lax.rsqrt(var + EPS) * lnw_ref[...]
                  + lnb_ref[...])


def _layernorm(psum, pcnt, ln_w, ln_b):
    return pl.pallas_call(
        _ln_body,
        in_specs=[
            pl.BlockSpec((NW, G, HL), lambda: (0, 0, 0)),
            pl.BlockSpec((NCHUNK, G, 16), lambda: (0, 0, 0)),
            pl.BlockSpec((1, D_OUT), lambda: (0, 0)),
            pl.BlockSpec((1, D_OUT), lambda: (0, 0)),
        ],
        out_specs=pl.BlockSpec((G, D_OUT), lambda: (0, 0)),
        out_shape=jax.ShapeDtypeStruct((G, D_OUT), jnp.float32),
    )(psum, pcnt, ln_w.reshape(1, D_OUT), ln_b.reshape(1, D_OUT))


def kernel(edge_attr, batch, W, b, ln_w, ln_b):
    h, gsum = _matmul_relu(edge_attr, W, b)
    psum, pcnt = _segsum(h.reshape(NH * E * HL), gsum.reshape(NH * EG * HL),
                         batch.astype(jnp.int32))
    return _layernorm(psum, pcnt, ln_w, ln_b)
